# baseline (speedup 1.0000x reference)
import jax
import jax.numpy as jnp
from jax import lax
from jax.experimental import pallas as pl
from jax.experimental.pallas import tpu as pltpu

_B, _C, _H, _W = 2, 96, 128, 128
_MID = 512
_NOUT = 18 + 36
_ROWS = 8
_SLAB = (_ROWS + 2) * _W


def _rpn_body(x_ref, w1_ref, b1_ref, wc_ref, bc_ref, out_ref):
    i = pl.program_id(1)
    slab = x_ref[0, :, pl.ds(i * _ROWS, _ROWS + 2), :].astype(jnp.bfloat16)
    xt = jnp.transpose(slab, (1, 2, 0)).reshape(_SLAB, _C)

    w_idx = lax.broadcasted_iota(jnp.int32, (_SLAB, 1), 0) & (_W - 1)
    zero = jnp.zeros_like(xt)
    xs = (
        jnp.where(w_idx == 0, zero, pltpu.roll(xt, 1, 0)),
        xt,
        jnp.where(w_idx == _W - 1, zero, pltpu.roll(xt, _SLAB - 1, 0)),
    )

    acc = jnp.zeros((_ROWS * _W, _MID), jnp.float32)
    for di in range(3):
        for dj in range(3):
            lhs = xs[dj][di * _W : di * _W + _ROWS * _W, :]
            acc += jnp.dot(lhs, w1_ref[di * 3 + dj],
                           preferred_element_type=jnp.float32)

    h = jnp.maximum(acc + b1_ref[0], 0.0).astype(jnp.bfloat16)
    out = jnp.dot(h, wc_ref[...], preferred_element_type=jnp.float32) + bc_ref[0]
    out_ref[...] = out.reshape(1, _ROWS, _W, _NOUT)


def kernel(features, W1, b1, W_obj, b_obj, W_tr, b_tr):
    xh = jnp.pad(features, ((0, 0), (0, 0), (1, 1), (0, 0)))
    w1m = (
        jnp.transpose(W1, (2, 3, 1, 0)).reshape(9, _C, _MID).astype(jnp.bfloat16)
    )
    wc = jnp.concatenate(
        [W_obj.reshape(18, _MID).T, W_tr.reshape(36, _MID).T], axis=1
    ).astype(jnp.bfloat16)
    bc = jnp.concatenate([b_obj, b_tr]).reshape(1, _NOUT)
    b1m = b1.reshape(1, _MID)

    out = pl.pallas_call(
        _rpn_body,
        grid=(_B, _H // _ROWS),
        in_specs=[
            pl.BlockSpec((1, _C, _H + 2, _W), lambda b, i: (b, 0, 0, 0)),
            pl.BlockSpec((9, _C, _MID), lambda b, i: (0, 0, 0)),
            pl.BlockSpec((1, _MID), lambda b, i: (0, 0)),
            pl.BlockSpec((_MID, _NOUT), lambda b, i: (0, 0)),
            pl.BlockSpec((1, _NOUT), lambda b, i: (0, 0)),
        ],
        out_specs=pl.BlockSpec((1, _ROWS, _W, _NOUT), lambda b, i: (b, i, 0, 0)),
        out_shape=jax.ShapeDtypeStruct((_B, _H, _W, _NOUT), jnp.float32),
        compiler_params=pltpu.CompilerParams(
            dimension_semantics=("parallel", "arbitrary"),
        ),
    )(xh, w1m, b1m, wc, bc)

    obj = out[..., :18].reshape(_B, -1, 2)
    tr = out[..., 18:].reshape(_B, -1, 4)
    return (obj, tr)


# speedup vs baseline: 1.2922x; 1.2922x over previous
import jax
import jax.numpy as jnp
from jax import lax
from jax.experimental import pallas as pl
from jax.experimental.pallas import tpu as pltpu

_B, _C, _H, _W = 2, 96, 128, 128
_MID = 512
_NOUT = 18 + 36
_ROWS = 8
_SLAB = (_ROWS + 2) * _W


def _rpn_body(x_ref, w1_ref, b1_ref, wc_ref, bc_ref, obj_ref, tr_ref):
    i = pl.program_id(1)
    slab = x_ref[0, :, pl.ds(i * _ROWS, _ROWS + 2), :].astype(jnp.bfloat16)
    xt = jnp.transpose(slab, (1, 2, 0)).reshape(_SLAB, _C)

    w_idx = lax.broadcasted_iota(jnp.int32, (_SLAB, 1), 0) & (_W - 1)
    zero = jnp.zeros_like(xt)
    xs = (
        jnp.where(w_idx == 0, zero, pltpu.roll(xt, 1, 0)),
        xt,
        jnp.where(w_idx == _W - 1, zero, pltpu.roll(xt, _SLAB - 1, 0)),
    )

    acc = jnp.zeros((_ROWS * _W, _MID), jnp.float32)
    for di in range(3):
        for dj in range(3):
            lhs = xs[dj][di * _W : di * _W + _ROWS * _W, :]
            acc += jnp.dot(lhs, w1_ref[di * 3 + dj],
                           preferred_element_type=jnp.float32)

    h = jnp.maximum(acc + b1_ref[0], 0.0).astype(jnp.bfloat16)
    out = jnp.dot(h, wc_ref[...], preferred_element_type=jnp.float32) + bc_ref[0]
    n = _ROWS * _W
    for a in range(9):
        obj_ref[0, pl.Slice(a, n, 9), :] = out[:, 2 * a : 2 * a + 2]
        tr_ref[0, pl.Slice(a, n, 9), :] = out[:, 18 + 4 * a : 18 + 4 * a + 4]


def kernel(features, W1, b1, W_obj, b_obj, W_tr, b_tr):
    xh = jnp.pad(features, ((0, 0), (0, 0), (1, 1), (0, 0)))
    w1m = (
        jnp.transpose(W1, (2, 3, 1, 0)).reshape(9, _C, _MID).astype(jnp.bfloat16)
    )
    wc = jnp.concatenate(
        [W_obj.reshape(18, _MID).T, W_tr.reshape(36, _MID).T], axis=1
    ).astype(jnp.bfloat16)
    bc = jnp.concatenate([b_obj, b_tr]).reshape(1, _NOUT)
    b1m = b1.reshape(1, _MID)

    out = pl.pallas_call(
        _rpn_body,
        grid=(_B, _H // _ROWS),
        in_specs=[
            pl.BlockSpec((1, _C, _H + 2, _W), lambda b, i: (b, 0, 0, 0)),
            pl.BlockSpec((9, _C, _MID), lambda b, i: (0, 0, 0)),
            pl.BlockSpec((1, _MID), lambda b, i: (0, 0)),
            pl.BlockSpec((_MID, _NOUT), lambda b, i: (0, 0)),
            pl.BlockSpec((1, _NOUT), lambda b, i: (0, 0)),
        ],
        out_specs=[
            pl.BlockSpec((1, _ROWS * _W * 9, 2), lambda b, i: (b, i, 0)),
            pl.BlockSpec((1, _ROWS * _W * 9, 4), lambda b, i: (b, i, 0)),
        ],
        out_shape=[
            jax.ShapeDtypeStruct((_B, _H * _W * 9, 2), jnp.float32),
            jax.ShapeDtypeStruct((_B, _H * _W * 9, 4), jnp.float32),
        ],
        compiler_params=pltpu.CompilerParams(
            dimension_semantics=("parallel", "arbitrary"),
        ),
    )(xh, w1m, b1m, wc, bc)

    obj, tr = out
    return (obj, tr)


# speedup vs baseline: 1.4912x; 1.1540x over previous
import jax
import jax.numpy as jnp
from jax import lax
from jax.experimental import pallas as pl
from jax.experimental.pallas import tpu as pltpu

_B, _C, _H, _W = 2, 96, 128, 128
_HW = _H * _W
_MID = 512
_NOUT = 18 + 36
_ROWS = 8
_PIX = _ROWS * _W
_SLAB = _PIX + 2 * _W


def _rpn_body(x_ref, w1_ref, b1_ref, wc_ref, bc_ref, qo_ref, qt_ref):
    i = pl.program_id(1)
    start = pl.multiple_of(i * _PIX, _W)
    slab = x_ref[0, :, pl.ds(start, _SLAB)]

    w_idx = lax.broadcasted_iota(jnp.int32, (_C, _SLAB), 1) & (_W - 1)
    zcol = jnp.zeros((_C, 1), jnp.bfloat16)
    left = jnp.concatenate([zcol, slab[:, : _SLAB - 1]], axis=1)
    right = jnp.concatenate([slab[:, 1:], zcol], axis=1)
    xs = (
        jnp.where(w_idx == 0, jnp.bfloat16(0), left),
        slab,
        jnp.where(w_idx == _W - 1, jnp.bfloat16(0), right),
    )

    acc = jnp.zeros((_MID, _PIX), jnp.float32)
    for di in range(3):
        for dj in range(3):
            rhs = xs[dj][:, di * _W : di * _W + _PIX]
            acc += jnp.dot(w1_ref[di * 3 + dj], rhs,
                           preferred_element_type=jnp.float32)

    hT = jnp.maximum(acc + b1_ref[...], 0.0).astype(jnp.bfloat16)
    q = jnp.dot(wc_ref[...], hT, preferred_element_type=jnp.float32) + bc_ref[...]
    qo_ref[...] = q[:18][None]
    qt_ref[...] = q[18:][None]


def kernel(features, W1, b1, W_obj, b_obj, W_tr, b_tr):
    xf = jnp.pad(
        features.reshape(_B, _C, _HW), ((0, 0), (0, 0), (_W, _W))
    ).astype(jnp.bfloat16)
    w1t = jnp.transpose(W1, (2, 3, 0, 1)).reshape(9, _MID, _C).astype(jnp.bfloat16)
    wc2 = jnp.concatenate(
        [W_obj.reshape(18, _MID), W_tr.reshape(36, _MID)], axis=0
    ).astype(jnp.bfloat16)
    b1c = b1.reshape(_MID, 1)
    bc2 = jnp.concatenate([b_obj, b_tr]).reshape(_NOUT, 1)

    qo, qt = pl.pallas_call(
        _rpn_body,
        grid=(_B, _H // _ROWS),
        in_specs=[
            pl.BlockSpec((1, _C, _HW + 2 * _W), lambda b, i: (b, 0, 0)),
            pl.BlockSpec((9, _MID, _C), lambda b, i: (0, 0, 0)),
            pl.BlockSpec((_MID, 1), lambda b, i: (0, 0)),
            pl.BlockSpec((_NOUT, _MID), lambda b, i: (0, 0)),
            pl.BlockSpec((_NOUT, 1), lambda b, i: (0, 0)),
        ],
        out_specs=[
            pl.BlockSpec((1, 18, _PIX), lambda b, i: (b, 0, i)),
            pl.BlockSpec((1, 36, _PIX), lambda b, i: (b, 0, i)),
        ],
        out_shape=[
            jax.ShapeDtypeStruct((_B, 18, _HW), jnp.float32),
            jax.ShapeDtypeStruct((_B, 36, _HW), jnp.float32),
        ],
        compiler_params=pltpu.CompilerParams(
            dimension_semantics=("parallel", "arbitrary"),
        ),
    )(xf, w1t, b1c, wc2, bc2)

    obj = jnp.transpose(qo.reshape(_B, 9, 2, _HW), (0, 3, 1, 2)).reshape(_B, -1, 2)
    tr = jnp.transpose(qt.reshape(_B, 9, 4, _HW), (0, 3, 1, 2)).reshape(_B, -1, 4)
    return (obj, tr)


# speedup vs baseline: 3.1044x; 2.0819x over previous
import jax
import jax.numpy as jnp
from jax import lax
from jax.experimental import pallas as pl
from jax.experimental.pallas import tpu as pltpu

_B, _C, _H, _W = 2, 96, 128, 128
_HW = _H * _W
_MID = 512
_NOUT = 18 + 36
_A = 9
_ROWS = 8
_PIX = _ROWS * _W
_SLAB = _PIX + 2 * _W
_NWIN = _PIX * _A // _W
_KW = 16


def _win_start(n):
    return min(_W * n // _A, _PIX - _KW)


def _rpn_body(x_ref, w1_ref, b1_ref, wc_ref, bc_ref, e_ref, am_ref,
              obj_ref, tr_ref):
    i = pl.program_id(1)
    start = pl.multiple_of(i * _PIX, _W)
    slab = x_ref[0, :, pl.ds(start, _SLAB)]

    w_idx = lax.broadcasted_iota(jnp.int32, (_C, _SLAB), 1) & (_W - 1)
    zcol = jnp.zeros((_C, 1), jnp.bfloat16)
    left = jnp.concatenate([zcol, slab[:, : _SLAB - 1]], axis=1)
    right = jnp.concatenate([slab[:, 1:], zcol], axis=1)
    xs = (
        jnp.where(w_idx == 0, jnp.bfloat16(0), left),
        slab,
        jnp.where(w_idx == _W - 1, jnp.bfloat16(0), right),
    )

    acc = jnp.zeros((_MID, _PIX), jnp.float32)
    for di in range(3):
        for dj in range(3):
            rhs = xs[dj][:, di * _W : di * _W + _PIX]
            acc += jnp.dot(w1_ref[di * 3 + dj], rhs,
                           preferred_element_type=jnp.float32)

    hT = jnp.maximum(acc + b1_ref[...], 0.0).astype(jnp.bfloat16)
    q = jnp.dot(wc_ref[...], hT, preferred_element_type=jnp.float32) + bc_ref[...]

    gw = []
    for n in range(_NWIN):
        p0 = _win_start(n)
        gw.append(jnp.dot(q[:, p0 : p0 + _KW], e_ref[n],
                          preferred_element_type=jnp.float32))
    g = jnp.concatenate(gw, axis=1)

    amod = am_ref[...]
    p2 = jnp.zeros((2, _PIX * _A), jnp.float32)
    p4 = jnp.zeros((4, _PIX * _A), jnp.float32)
    for a in range(_A):
        sel = amod == a
        p2 = p2 + jnp.where(sel, g[2 * a : 2 * a + 2, :], 0.0)
        p4 = p4 + jnp.where(sel, g[18 + 4 * a : 18 + 4 * a + 4, :], 0.0)
    obj_ref[...] = p2[None]
    tr_ref[...] = p4[None]


def kernel(features, W1, b1, W_obj, b_obj, W_tr, b_tr):
    xf = jnp.pad(
        features.reshape(_B, _C, _HW), ((0, 0), (0, 0), (_W, _W))
    ).astype(jnp.bfloat16)
    w1t = jnp.transpose(W1, (2, 3, 0, 1)).reshape(9, _MID, _C).astype(jnp.bfloat16)
    wc2 = jnp.concatenate(
        [W_obj.reshape(18, _MID), W_tr.reshape(36, _MID)], axis=0
    ).astype(jnp.bfloat16)
    b1c = b1.reshape(_MID, 1)
    bc2 = jnp.concatenate([b_obj, b_tr]).reshape(_NOUT, 1)

    nn = jnp.arange(_NWIN)[:, None, None]
    jj = jnp.arange(_KW)[None, :, None]
    tt = jnp.arange(_W)[None, None, :]
    p0 = jnp.minimum(_W * nn // _A, _PIX - _KW)
    eb = ((p0 + jj) == (_W * nn + tt) // _A).astype(jnp.float32)
    amod = (jnp.arange(_PIX * _A, dtype=jnp.int32) % _A).reshape(1, -1)

    po, pt = pl.pallas_call(
        _rpn_body,
        grid=(_B, _H // _ROWS),
        in_specs=[
            pl.BlockSpec((1, _C, _HW + 2 * _W), lambda b, i: (b, 0, 0)),
            pl.BlockSpec((9, _MID, _C), lambda b, i: (0, 0, 0)),
            pl.BlockSpec((_MID, 1), lambda b, i: (0, 0)),
            pl.BlockSpec((_NOUT, _MID), lambda b, i: (0, 0)),
            pl.BlockSpec((_NOUT, 1), lambda b, i: (0, 0)),
            pl.BlockSpec((_NWIN, _KW, _W), lambda b, i: (0, 0, 0)),
            pl.BlockSpec((1, _PIX * _A), lambda b, i: (0, 0)),
        ],
        out_specs=[
            pl.BlockSpec((1, 2, _PIX * _A), lambda b, i: (b, 0, i)),
            pl.BlockSpec((1, 4, _PIX * _A), lambda b, i: (b, 0, i)),
        ],
        out_shape=[
            jax.ShapeDtypeStruct((_B, 2, _HW * _A), jnp.float32),
            jax.ShapeDtypeStruct((_B, 4, _HW * _A), jnp.float32),
        ],
        compiler_params=pltpu.CompilerParams(
            dimension_semantics=("parallel", "arbitrary"),
        ),
    )(xf, w1t, b1c, wc2, bc2, eb, amod)

    obj = jnp.transpose(po, (0, 2, 1))
    tr = jnp.transpose(pt, (0, 2, 1))
    return (obj, tr)


# speedup vs baseline: 3.2598x; 1.0501x over previous
import jax
import jax.numpy as jnp
from jax import lax
from jax.experimental import pallas as pl
from jax.experimental.pallas import tpu as pltpu

_B, _C, _H, _W = 2, 96, 128, 128
_HW = _H * _W
_MID = 512
_NOUT = 18 + 36
_A = 9
_ROWS = 8
_PIX = _ROWS * _W
_SLAB = _PIX + 2 * _W
_NWIN = _PIX * _A // _W
_KW = 16


def _win_start(n):
    return min(_W * n // _A, _PIX - _KW)


def _rpn_body(x_ref, w1_ref, b1_ref, wc_ref, bc_ref, e_ref, am_ref,
              obj_ref, tr_ref):
    i = pl.program_id(1)
    start = pl.multiple_of(i * _PIX, _W)
    slab = x_ref[0, :, pl.ds(start, _SLAB)].astype(jnp.bfloat16)

    w_idx = lax.broadcasted_iota(jnp.int32, (_C, _SLAB), 1) & (_W - 1)
    zcol = jnp.zeros((_C, 1), jnp.bfloat16)
    left = jnp.concatenate([zcol, slab[:, : _SLAB - 1]], axis=1)
    right = jnp.concatenate([slab[:, 1:], zcol], axis=1)
    xs = (
        jnp.where(w_idx == 0, jnp.bfloat16(0), left),
        slab,
        jnp.where(w_idx == _W - 1, jnp.bfloat16(0), right),
    )

    acc = jnp.zeros((_MID, _PIX), jnp.float32)
    for di in range(3):
        for dj in range(3):
            rhs = xs[dj][:, di * _W : di * _W + _PIX]
            acc += jnp.dot(w1_ref[di * 3 + dj], rhs,
                           preferred_element_type=jnp.float32)

    hT = jnp.maximum(acc + b1_ref[...], 0.0).astype(jnp.bfloat16)
    q = jnp.dot(wc_ref[...], hT, preferred_element_type=jnp.float32) + bc_ref[...]

    qb = q.astype(jnp.bfloat16)
    gw = []
    for n in range(_NWIN):
        p0 = _win_start(n)
        gw.append(jnp.dot(qb[:, p0 : p0 + _KW], e_ref[n],
                          preferred_element_type=jnp.float32))
    g = jnp.concatenate(gw, axis=1)

    amod = am_ref[...]
    p2 = jnp.zeros((2, _PIX * _A), jnp.float32)
    p4 = jnp.zeros((4, _PIX * _A), jnp.float32)
    for a in range(_A):
        sel = amod == a
        p2 = p2 + jnp.where(sel, g[2 * a : 2 * a + 2, :], 0.0)
        p4 = p4 + jnp.where(sel, g[18 + 4 * a : 18 + 4 * a + 4, :], 0.0)
    obj_ref[...] = p2[None]
    tr_ref[...] = p4[None]


def kernel(features, W1, b1, W_obj, b_obj, W_tr, b_tr):
    xf = jnp.pad(
        features.reshape(_B, _C, _HW), ((0, 0), (0, 0), (_W, _W))
    )
    w1t = jnp.transpose(W1, (2, 3, 0, 1)).reshape(9, _MID, _C).astype(jnp.bfloat16)
    wc2 = jnp.concatenate(
        [W_obj.reshape(18, _MID), W_tr.reshape(36, _MID)], axis=0
    ).astype(jnp.bfloat16)
    b1c = b1.reshape(_MID, 1)
    bc2 = jnp.concatenate([b_obj, b_tr]).reshape(_NOUT, 1)

    nn = jnp.arange(_NWIN)[:, None, None]
    jj = jnp.arange(_KW)[None, :, None]
    tt = jnp.arange(_W)[None, None, :]
    p0 = jnp.minimum(_W * nn // _A, _PIX - _KW)
    eb = ((p0 + jj) == (_W * nn + tt) // _A).astype(jnp.bfloat16)
    amod = (jnp.arange(_PIX * _A, dtype=jnp.int32) % _A).reshape(1, -1)

    po, pt = pl.pallas_call(
        _rpn_body,
        grid=(_B, _H // _ROWS),
        in_specs=[
            pl.BlockSpec((1, _C, _HW + 2 * _W), lambda b, i: (b, 0, 0)),
            pl.BlockSpec((9, _MID, _C), lambda b, i: (0, 0, 0)),
            pl.BlockSpec((_MID, 1), lambda b, i: (0, 0)),
            pl.BlockSpec((_NOUT, _MID), lambda b, i: (0, 0)),
            pl.BlockSpec((_NOUT, 1), lambda b, i: (0, 0)),
            pl.BlockSpec((_NWIN, _KW, _W), lambda b, i: (0, 0, 0)),
            pl.BlockSpec((1, _PIX * _A), lambda b, i: (0, 0)),
        ],
        out_specs=[
            pl.BlockSpec((1, 2, _PIX * _A), lambda b, i: (b, 0, i)),
            pl.BlockSpec((1, 4, _PIX * _A), lambda b, i: (b, 0, i)),
        ],
        out_shape=[
            jax.ShapeDtypeStruct((_B, 2, _HW * _A), jnp.float32),
            jax.ShapeDtypeStruct((_B, 4, _HW * _A), jnp.float32),
        ],
        compiler_params=pltpu.CompilerParams(
            dimension_semantics=("parallel", "arbitrary"),
        ),
    )(xf, w1t, b1c, wc2, bc2, eb, amod)

    obj = jnp.transpose(po, (0, 2, 1))
    tr = jnp.transpose(pt, (0, 2, 1))
    return (obj, tr)


# speedup vs baseline: 3.2773x; 1.0054x over previous
import jax
import jax.numpy as jnp
from jax import lax
from jax.experimental import pallas as pl
from jax.experimental.pallas import tpu as pltpu

_B, _C, _H, _W = 2, 96, 128, 128
_HW = _H * _W
_MID = 512
_NOUT = 18 + 36
_A = 9
_ROWS = 16
_PIX = _ROWS * _W
_SLAB = _PIX + 2 * _W
_NWIN = _PIX * _A // _W
_KW = 16


def _win_start(n):
    return min(_W * n // _A, _PIX - _KW)


def _rpn_body(x_ref, w1_ref, b1_ref, wc_ref, bc_ref, e_ref, am_ref,
              obj_ref, tr_ref):
    i = pl.program_id(1)
    start = pl.multiple_of(i * _PIX, _W)
    slab = x_ref[0, :, pl.ds(start, _SLAB)].astype(jnp.bfloat16)

    w_idx = lax.broadcasted_iota(jnp.int32, (_C, _SLAB), 1) & (_W - 1)
    zcol = jnp.zeros((_C, 1), jnp.bfloat16)
    left = jnp.concatenate([zcol, slab[:, : _SLAB - 1]], axis=1)
    right = jnp.concatenate([slab[:, 1:], zcol], axis=1)
    xs = (
        jnp.where(w_idx == 0, jnp.bfloat16(0), left),
        slab,
        jnp.where(w_idx == _W - 1, jnp.bfloat16(0), right),
    )

    acc = jnp.zeros((_MID, _PIX), jnp.float32)
    for di in range(3):
        for dj in range(3):
            rhs = xs[dj][:, di * _W : di * _W + _PIX]
            acc += jnp.dot(w1_ref[di * 3 + dj], rhs,
                           preferred_element_type=jnp.float32)

    hT = jnp.maximum(acc + b1_ref[...], 0.0).astype(jnp.bfloat16)
    q = jnp.dot(wc_ref[...], hT, preferred_element_type=jnp.float32) + bc_ref[...]

    qb = q.astype(jnp.bfloat16)
    gw = []
    for n in range(_NWIN):
        p0 = _win_start(n)
        gw.append(jnp.dot(qb[:, p0 : p0 + _KW], e_ref[n],
                          preferred_element_type=jnp.float32))
    g = jnp.concatenate(gw, axis=1)

    amod = am_ref[...]
    p2 = jnp.zeros((2, _PIX * _A), jnp.float32)
    p4 = jnp.zeros((4, _PIX * _A), jnp.float32)
    for a in range(_A):
        sel = amod == a
        p2 = p2 + jnp.where(sel, g[2 * a : 2 * a + 2, :], 0.0)
        p4 = p4 + jnp.where(sel, g[18 + 4 * a : 18 + 4 * a + 4, :], 0.0)
    obj_ref[...] = p2[None]
    tr_ref[...] = p4[None]


def kernel(features, W1, b1, W_obj, b_obj, W_tr, b_tr):
    xf = jnp.pad(
        features.reshape(_B, _C, _HW), ((0, 0), (0, 0), (_W, _W))
    )
    w1t = jnp.transpose(W1, (2, 3, 0, 1)).reshape(9, _MID, _C).astype(jnp.bfloat16)
    wc2 = jnp.concatenate(
        [W_obj.reshape(18, _MID), W_tr.reshape(36, _MID)], axis=0
    ).astype(jnp.bfloat16)
    b1c = b1.reshape(_MID, 1)
    bc2 = jnp.concatenate([b_obj, b_tr]).reshape(_NOUT, 1)

    nn = jnp.arange(_NWIN)[:, None, None]
    jj = jnp.arange(_KW)[None, :, None]
    tt = jnp.arange(_W)[None, None, :]
    p0 = jnp.minimum(_W * nn // _A, _PIX - _KW)
    eb = ((p0 + jj) == (_W * nn + tt) // _A).astype(jnp.bfloat16)
    amod = (jnp.arange(_PIX * _A, dtype=jnp.int32) % _A).reshape(1, -1)

    po, pt = pl.pallas_call(
        _rpn_body,
        grid=(_B, _H // _ROWS),
        in_specs=[
            pl.BlockSpec((1, _C, _HW + 2 * _W), lambda b, i: (b, 0, 0)),
            pl.BlockSpec((9, _MID, _C), lambda b, i: (0, 0, 0)),
            pl.BlockSpec((_MID, 1), lambda b, i: (0, 0)),
            pl.BlockSpec((_NOUT, _MID), lambda b, i: (0, 0)),
            pl.BlockSpec((_NOUT, 1), lambda b, i: (0, 0)),
            pl.BlockSpec((_NWIN, _KW, _W), lambda b, i: (0, 0, 0)),
            pl.BlockSpec((1, _PIX * _A), lambda b, i: (0, 0)),
        ],
        out_specs=[
            pl.BlockSpec((1, 2, _PIX * _A), lambda b, i: (b, 0, i)),
            pl.BlockSpec((1, 4, _PIX * _A), lambda b, i: (b, 0, i)),
        ],
        out_shape=[
            jax.ShapeDtypeStruct((_B, 2, _HW * _A), jnp.float32),
            jax.ShapeDtypeStruct((_B, 4, _HW * _A), jnp.float32),
        ],
        compiler_params=pltpu.CompilerParams(
            dimension_semantics=("parallel", "arbitrary"),
        ),
    )(xf, w1t, b1c, wc2, bc2, eb, amod)

    obj = jnp.transpose(po, (0, 2, 1))
    tr = jnp.transpose(pt, (0, 2, 1))
    return (obj, tr)
